# baseline (speedup 1.0000x reference)
import functools

import jax
import jax.numpy as jnp
from jax import lax
from jax.experimental import pallas as pl
from jax.experimental.pallas import tpu as pltpu
from jax.experimental.pallas import tpu_sc as plsc

N = 10000
E = 320000
D = 128
G = 64
L = 3

NC = 2
NS = 16
NW = NC * NS
CHUNK = 80
NCHUNK = 128
EPW = NCHUNK * CHUNK
EPAD = NW * EPW - E
NBLK = 8
BLK = NCHUNK // NBLK
NDUM = 16
NACC = N + NDUM
ROWS_PER_SUB = 624
TAIL_OFF = NS * ROWS_PER_SUB
ROWS_TAIL = N - TAIL_OFF
ZTAIL = NACC - TAIL_OFF
EPS = 1e-5

def _agg_impl(h_hbm, src_hbm, dst_hbm, zeros_hbm, out_hbm,
              src_v, dst_v, rows0, rows1, rows2, rows3, acc,
              gsem0, gsem1, gsem2, gsem3, ssem0, ssem1, ssem2, ssem3):
    c = lax.axis_index("c")
    s = lax.axis_index("s")
    wid = c * NS + s

    pltpu.sync_copy(zeros_hbm.at[pl.ds(s * ROWS_PER_SUB, ROWS_PER_SUB)],
                    acc.at[pl.ds(s * ROWS_PER_SUB, ROWS_PER_SUB)])
    @pl.when(s == 0)
    def _():
        pltpu.sync_copy(zeros_hbm.at[pl.ds(TAIL_OFF, ZTAIL)],
                        acc.at[pl.ds(TAIL_OFF, ZTAIL)])
    plsc.subcore_barrier()

    def blk_body(blk, carry):
        pltpu.sync_copy(src_hbm.at[wid, blk], src_v)
        pltpu.sync_copy(dst_hbm.at[wid, blk], dst_v)

        rows = (rows0, rows1, rows2, rows3)
        gsems = (gsem0, gsem1, gsem2, gsem3)
        ssems = (ssem0, ssem1, ssem2, ssem3)

        def body(q, carry2):
            cps = [pltpu.async_copy(h_hbm.at[src_v.at[4 * q + u]], rows[u],
                                    gsems[u]) for u in range(4)]
            scs = []
            for u in range(4):
                cps[u].wait()
                scs.append(pltpu.async_copy(rows[u], acc.at[dst_v.at[4 * q + u]],
                                            ssems[u], add=True))
            for sc in scs:
                sc.wait()
            return carry2

        lax.fori_loop(0, BLK // 4, body, 0)
        return carry

    lax.fori_loop(0, NBLK, blk_body, 0)

    plsc.subcore_barrier()
    pltpu.sync_copy(acc.at[pl.ds(s * ROWS_PER_SUB, ROWS_PER_SUB)],
                    out_hbm.at[pl.ds(c * N + s * ROWS_PER_SUB, ROWS_PER_SUB)])
    @pl.when(s == 0)
    def _():
        pltpu.sync_copy(acc.at[pl.ds(TAIL_OFF, ROWS_TAIL)],
                        out_hbm.at[pl.ds(c * N + TAIL_OFF, ROWS_TAIL)])


@functools.lru_cache(maxsize=None)
def _get_agg():
    mesh = plsc.VectorSubcoreMesh(core_axis_name="c", subcore_axis_name="s")
    return pl.kernel(
        _agg_impl,
        out_type=jax.ShapeDtypeStruct((NC * N, D), jnp.float32),
        mesh=mesh,
        scratch_types=[
            pltpu.VMEM((BLK, CHUNK), jnp.int32),
            pltpu.VMEM((BLK, CHUNK), jnp.int32),
            pltpu.VMEM((CHUNK, D), jnp.float32),
            pltpu.VMEM((CHUNK, D), jnp.float32),
            pltpu.VMEM((CHUNK, D), jnp.float32),
            pltpu.VMEM((CHUNK, D), jnp.float32),
            pltpu.VMEM_SHARED((NACC, D), jnp.float32),
        ] + [pltpu.SemaphoreType.DMA] * 8,
    )


def _bn_relu(u, gamma, beta):
    mean = jnp.mean(u, axis=0, keepdims=True)
    var = jnp.mean(jnp.square(u - mean), axis=0, keepdims=True)
    return jnp.maximum((u - mean) * lax.rsqrt(var + EPS) * gamma + beta, 0.0)


def _mlp_common(h_ref, a_ref, w1_ref, b1_ref, g1_ref, be1_ref,
                w2_ref, b2_ref, g2_ref, be2_ref):
    t = h_ref[...] + a_ref[:N] + a_ref[N:]
    u = jnp.dot(t, w1_ref[...], preferred_element_type=jnp.float32) + b1_ref[...]
    u = _bn_relu(u, g1_ref[...], be1_ref[...])
    v = jnp.dot(u, w2_ref[...], preferred_element_type=jnp.float32) + b2_ref[...]
    return _bn_relu(v, g2_ref[...], be2_ref[...])


def _mlp_body(h_ref, a_ref, w1_ref, b1_ref, g1_ref, be1_ref,
              w2_ref, b2_ref, g2_ref, be2_ref, o_ref):
    o_ref[...] = _mlp_common(h_ref, a_ref, w1_ref, b1_ref, g1_ref, be1_ref,
                             w2_ref, b2_ref, g2_ref, be2_ref)


def _mlp_pool_body(h_ref, a_ref, batch_ref, w1_ref, b1_ref, g1_ref, be1_ref,
                   w2_ref, b2_ref, g2_ref, be2_ref, o_ref):
    hout = _mlp_common(h_ref, a_ref, w1_ref, b1_ref, g1_ref, be1_ref,
                       w2_ref, b2_ref, g2_ref, be2_ref)
    gids = lax.broadcasted_iota(jnp.int32, (N, G), 1)
    onehot = (batch_ref[...] == gids).astype(jnp.float32)
    o_ref[...] = lax.dot_general(onehot, hout, (((0,), (0,)), ((), ())),
                                 preferred_element_type=jnp.float32)


_mlp = pl.pallas_call(_mlp_body, out_shape=jax.ShapeDtypeStruct((N, D), jnp.float32))
_mlp_pool = pl.pallas_call(_mlp_pool_body, out_shape=jax.ShapeDtypeStruct((G, D), jnp.float32))


def kernel(x, edge_index, batch, W1, b1, g1, beta1, W2, b2, g2, beta2):
    pad_src = jnp.zeros((EPAD,), jnp.int32)
    pad_dst = N + (jnp.arange(EPAD, dtype=jnp.int32) % NDUM)
    src = jnp.concatenate([edge_index[0], pad_src]).reshape(NW, NBLK, BLK, CHUNK)
    dst = jnp.concatenate([edge_index[1], pad_dst]).reshape(NW, NBLK, BLK, CHUNK)
    zeros = jnp.zeros((NACC, D), jnp.float32)
    batch2 = batch.reshape(N, 1)
    agg = _get_agg()
    h = x
    for i in range(L):
        agg2 = agg(h, src, dst, zeros)
        params = (W1[i], b1[i].reshape(1, D), g1[i].reshape(1, D),
                  beta1[i].reshape(1, D), W2[i], b2[i].reshape(1, D),
                  g2[i].reshape(1, D), beta2[i].reshape(1, D))
        if i < L - 1:
            h = _mlp(h, agg2, *params)
        else:
            out = _mlp_pool(h, agg2, batch2, *params)
    return out


# speedup vs baseline: 2.7324x; 2.7324x over previous
import functools

import jax
import jax.numpy as jnp
from jax import lax
from jax.experimental import pallas as pl
from jax.experimental.pallas import tpu as pltpu
from jax.experimental.pallas import tpu_sc as plsc

N = 10000
E = 320000
D = 128
G = 64
L = 3

NC = 2
NS = 16
NW = NC * NS
CHUNK = 80
NCHUNK = 128
EPW = NCHUNK * CHUNK
EPAD = NW * EPW - E
NBLK = 8
BLK = NCHUNK // NBLK
NDUM = 16
NACC = N + NDUM
ROWS_PER_SUB = 624
TAIL_OFF = NS * ROWS_PER_SUB
ROWS_TAIL = N - TAIL_OFF
ZTAIL = NACC - TAIL_OFF
EPS = 1e-5

def _agg_impl(h_hbm, src_hbm, dst_hbm, zeros_hbm, out_hbm,
              src_v, dst_v, rows0, rows1, rows2, rows3, acc,
              gsem0, gsem1, gsem2, gsem3, ssem0, ssem1, ssem2, ssem3):
    c = lax.axis_index("c")
    s = lax.axis_index("s")
    wid = c * NS + s

    pltpu.sync_copy(zeros_hbm.at[pl.ds(s * ROWS_PER_SUB, ROWS_PER_SUB)],
                    acc.at[pl.ds(s * ROWS_PER_SUB, ROWS_PER_SUB)])
    @pl.when(s == 0)
    def _():
        pltpu.sync_copy(zeros_hbm.at[pl.ds(TAIL_OFF, ZTAIL)],
                        acc.at[pl.ds(TAIL_OFF, ZTAIL)])
    plsc.subcore_barrier()

    def blk_body(blk, carry):
        pltpu.sync_copy(src_hbm.at[wid, blk], src_v)
        pltpu.sync_copy(dst_hbm.at[wid, blk], dst_v)

        rows = (rows0, rows1, rows2, rows3)
        gsems = (gsem0, gsem1, gsem2, gsem3)
        ssems = (ssem0, ssem1, ssem2, ssem3)

        def body(q, carry2):
            cps = [pltpu.async_copy(h_hbm.at[src_v.at[4 * q + u]], rows[u],
                                    gsems[u]) for u in range(4)]
            scs = []
            for u in range(4):
                cps[u].wait()
                scs.append(pltpu.async_copy(rows[u], acc.at[dst_v.at[4 * q + u]],
                                            ssems[u], add=True))
            for sc in scs:
                sc.wait()
            return carry2

        lax.fori_loop(0, BLK // 4, body, 0)
        return carry

    lax.fori_loop(0, NBLK, blk_body, 0)

    plsc.subcore_barrier()
    pltpu.sync_copy(acc.at[pl.ds(s * ROWS_PER_SUB, ROWS_PER_SUB)],
                    out_hbm.at[pl.ds(c * N + s * ROWS_PER_SUB, ROWS_PER_SUB)])
    @pl.when(s == 0)
    def _():
        pltpu.sync_copy(acc.at[pl.ds(TAIL_OFF, ROWS_TAIL)],
                        out_hbm.at[pl.ds(c * N + TAIL_OFF, ROWS_TAIL)])


@functools.lru_cache(maxsize=None)
def _get_agg():
    mesh = plsc.VectorSubcoreMesh(core_axis_name="c", subcore_axis_name="s")
    return pl.kernel(
        _agg_impl,
        out_type=jax.ShapeDtypeStruct((NC * N, D), jnp.float32),
        mesh=mesh,
        scratch_types=[
            pltpu.VMEM((BLK, CHUNK), jnp.int32),
            pltpu.VMEM((BLK, CHUNK), jnp.int32),
            pltpu.VMEM((CHUNK, D), jnp.float32),
            pltpu.VMEM((CHUNK, D), jnp.float32),
            pltpu.VMEM((CHUNK, D), jnp.float32),
            pltpu.VMEM((CHUNK, D), jnp.float32),
            pltpu.VMEM_SHARED((NACC, D), jnp.float32),
        ] + [pltpu.SemaphoreType.DMA] * 8,
    )


def _bn_relu(u, gamma, beta):
    mean = jnp.mean(u, axis=0, keepdims=True)
    var = jnp.mean(jnp.square(u - mean), axis=0, keepdims=True)
    return jnp.maximum((u - mean) * lax.rsqrt(var + EPS) * gamma + beta, 0.0)


def _mlp_common(h_ref, a_ref, w1_ref, b1_ref, g1_ref, be1_ref,
                w2_ref, b2_ref, g2_ref, be2_ref):
    t = h_ref[...] + a_ref[:N] + a_ref[N:]
    u = jnp.dot(t, w1_ref[...], preferred_element_type=jnp.float32) + b1_ref[...]
    u = _bn_relu(u, g1_ref[...], be1_ref[...])
    v = jnp.dot(u, w2_ref[...], preferred_element_type=jnp.float32) + b2_ref[...]
    return _bn_relu(v, g2_ref[...], be2_ref[...])


def _mlp_body(h_ref, a_ref, w1_ref, b1_ref, g1_ref, be1_ref,
              w2_ref, b2_ref, g2_ref, be2_ref, o_ref):
    o_ref[...] = _mlp_common(h_ref, a_ref, w1_ref, b1_ref, g1_ref, be1_ref,
                             w2_ref, b2_ref, g2_ref, be2_ref)


def _mlp_pool_body(h_ref, a_ref, batch_ref, w1_ref, b1_ref, g1_ref, be1_ref,
                   w2_ref, b2_ref, g2_ref, be2_ref, o_ref):
    hout = _mlp_common(h_ref, a_ref, w1_ref, b1_ref, g1_ref, be1_ref,
                       w2_ref, b2_ref, g2_ref, be2_ref)
    gids = lax.broadcasted_iota(jnp.int32, (N, G), 1)
    onehot = (batch_ref[...] == gids).astype(jnp.float32)
    o_ref[...] = lax.dot_general(onehot, hout, (((0,), (0,)), ((), ())),
                                 preferred_element_type=jnp.float32)


_mlp = pl.pallas_call(_mlp_body, out_shape=jax.ShapeDtypeStruct((N, D), jnp.float32))
_mlp_pool = pl.pallas_call(_mlp_pool_body, out_shape=jax.ShapeDtypeStruct((G, D), jnp.float32))


def kernel(x, edge_index, batch, W1, b1, g1, beta1, W2, b2, g2, beta2):
    ppw = EPW - E // NW
    pad_src = (jnp.arange(NW * ppw, dtype=jnp.int32) * 13) % N
    pad_dst = N + (jnp.arange(NW * ppw, dtype=jnp.int32) % NDUM)
    src = jnp.concatenate([edge_index[0].reshape(NW, E // NW),
                           pad_src.reshape(NW, ppw)], axis=1)
    dst = jnp.concatenate([edge_index[1].reshape(NW, E // NW),
                           pad_dst.reshape(NW, ppw)], axis=1)
    src = src.reshape(NW, NBLK, BLK, CHUNK)
    dst = dst.reshape(NW, NBLK, BLK, CHUNK)
    zeros = jnp.zeros((NACC, D), jnp.float32)
    batch2 = batch.reshape(N, 1)
    agg = _get_agg()
    h = x
    for i in range(L):
        agg2 = agg(h, src, dst, zeros)
        params = (W1[i], b1[i].reshape(1, D), g1[i].reshape(1, D),
                  beta1[i].reshape(1, D), W2[i], b2[i].reshape(1, D),
                  g2[i].reshape(1, D), beta2[i].reshape(1, D))
        if i < L - 1:
            h = _mlp(h, agg2, *params)
        else:
            out = _mlp_pool(h, agg2, batch2, *params)
    return out


# speedup vs baseline: 3.8276x; 1.4008x over previous
import functools

import jax
import jax.numpy as jnp
from jax import lax
from jax.experimental import pallas as pl
from jax.experimental.pallas import tpu as pltpu
from jax.experimental.pallas import tpu_sc as plsc

N = 10000
E = 320000
D = 128
G = 64
L = 3

NC = 2
NS = 16
NW = NC * NS
CHUNK = 56
NCHUNK = 180
EPW = NCHUNK * CHUNK
NDUM = 16
NACC = N + NDUM
ROWS_PER_SUB = 624
TAIL_OFF = NS * ROWS_PER_SUB
ROWS_TAIL = N - TAIL_OFF
ZTAIL = NACC - TAIL_OFF
EPS = 1e-5

def _agg_impl(h_hbm, src_hbm, dst_hbm, zeros_hbm, out_hbm,
              src_v, dst_v, rows0, rows1, rows2, rows3, acc,
              gsem0, gsem1, gsem2, gsem3, ssem0, ssem1, ssem2, ssem3):
    c = lax.axis_index("c")
    s = lax.axis_index("s")
    wid = c * NS + s

    pltpu.sync_copy(zeros_hbm.at[pl.ds(s * ROWS_PER_SUB, ROWS_PER_SUB)],
                    acc.at[pl.ds(s * ROWS_PER_SUB, ROWS_PER_SUB)])
    @pl.when(s == 0)
    def _():
        pltpu.sync_copy(zeros_hbm.at[pl.ds(TAIL_OFF, ZTAIL)],
                        acc.at[pl.ds(TAIL_OFF, ZTAIL)])
    plsc.subcore_barrier()

    pltpu.sync_copy(src_hbm.at[wid], src_v)
    pltpu.sync_copy(dst_hbm.at[wid], dst_v)

    rows = (rows0, rows1, rows2, rows3)
    gsems = (gsem0, gsem1, gsem2, gsem3)
    ssems = (ssem0, ssem1, ssem2, ssem3)

    def g_desc(j, u):
        idx = src_v.at[pl.ds(j * CHUNK, CHUNK)]
        return pltpu.make_async_copy(h_hbm.at[idx], rows[u], gsems[u])

    def s_desc(j, u):
        idx = dst_v.at[pl.ds(j * CHUNK, CHUNK)]
        return pltpu.make_async_copy(rows[u], acc.at[idx], ssems[u])

    def issue_scatter(j, u):
        idx = dst_v.at[pl.ds(j * CHUNK, CHUNK)]
        pltpu.async_copy(rows[u], acc.at[idx], ssems[u], add=True)

    g_desc(0, 0).start()
    g_desc(1, 1).start()

    def body(i, carry):
        for u in range(4):
            j = 4 * i + u
            v = (u + 2) % 4

            @pl.when(j >= 2)
            def _():
                s_desc(j - 2, v).wait()

            @pl.when(j + 2 < NCHUNK)
            def _():
                g_desc(j + 2, v).start()

            g_desc(j, u).wait()
            issue_scatter(j, u)
        return carry

    lax.fori_loop(0, NCHUNK // 4, body, 0)
    s_desc(NCHUNK - 2, 2).wait()
    s_desc(NCHUNK - 1, 3).wait()

    plsc.subcore_barrier()
    pltpu.sync_copy(acc.at[pl.ds(s * ROWS_PER_SUB, ROWS_PER_SUB)],
                    out_hbm.at[pl.ds(c * N + s * ROWS_PER_SUB, ROWS_PER_SUB)])
    @pl.when(s == 0)
    def _():
        pltpu.sync_copy(acc.at[pl.ds(TAIL_OFF, ROWS_TAIL)],
                        out_hbm.at[pl.ds(c * N + TAIL_OFF, ROWS_TAIL)])


@functools.lru_cache(maxsize=None)
def _get_agg():
    mesh = plsc.VectorSubcoreMesh(core_axis_name="c", subcore_axis_name="s")
    return pl.kernel(
        _agg_impl,
        out_type=jax.ShapeDtypeStruct((NC * N, D), jnp.float32),
        mesh=mesh,
        scratch_types=[
            pltpu.VMEM((EPW,), jnp.int32),
            pltpu.VMEM((EPW,), jnp.int32),
            pltpu.VMEM((CHUNK, D), jnp.float32),
            pltpu.VMEM((CHUNK, D), jnp.float32),
            pltpu.VMEM((CHUNK, D), jnp.float32),
            pltpu.VMEM((CHUNK, D), jnp.float32),
            pltpu.VMEM_SHARED((NACC, D), jnp.float32),
        ] + [pltpu.SemaphoreType.DMA] * 8,
    )


def _bn_relu(u, gamma, beta):
    mean = jnp.mean(u, axis=0, keepdims=True)
    var = jnp.mean(jnp.square(u - mean), axis=0, keepdims=True)
    return jnp.maximum((u - mean) * lax.rsqrt(var + EPS) * gamma + beta, 0.0)


def _mlp_common(h_ref, a_ref, w1_ref, b1_ref, g1_ref, be1_ref,
                w2_ref, b2_ref, g2_ref, be2_ref):
    t = h_ref[...] + a_ref[:N] + a_ref[N:]
    u = jnp.dot(t, w1_ref[...], preferred_element_type=jnp.float32) + b1_ref[...]
    u = _bn_relu(u, g1_ref[...], be1_ref[...])
    v = jnp.dot(u, w2_ref[...], preferred_element_type=jnp.float32) + b2_ref[...]
    return _bn_relu(v, g2_ref[...], be2_ref[...])


def _mlp_body(h_ref, a_ref, w1_ref, b1_ref, g1_ref, be1_ref,
              w2_ref, b2_ref, g2_ref, be2_ref, o_ref):
    o_ref[...] = _mlp_common(h_ref, a_ref, w1_ref, b1_ref, g1_ref, be1_ref,
                             w2_ref, b2_ref, g2_ref, be2_ref)


def _mlp_pool_body(h_ref, a_ref, batch_ref, w1_ref, b1_ref, g1_ref, be1_ref,
                   w2_ref, b2_ref, g2_ref, be2_ref, o_ref):
    hout = _mlp_common(h_ref, a_ref, w1_ref, b1_ref, g1_ref, be1_ref,
                       w2_ref, b2_ref, g2_ref, be2_ref)
    gids = lax.broadcasted_iota(jnp.int32, (N, G), 1)
    onehot = (batch_ref[...] == gids).astype(jnp.float32)
    o_ref[...] = lax.dot_general(onehot, hout, (((0,), (0,)), ((), ())),
                                 preferred_element_type=jnp.float32)


_mlp = pl.pallas_call(_mlp_body, out_shape=jax.ShapeDtypeStruct((N, D), jnp.float32))
_mlp_pool = pl.pallas_call(_mlp_pool_body, out_shape=jax.ShapeDtypeStruct((G, D), jnp.float32))


def kernel(x, edge_index, batch, W1, b1, g1, beta1, W2, b2, g2, beta2):
    ppw = EPW - E // NW
    pad_src = (jnp.arange(NW * ppw, dtype=jnp.int32) * 13) % N
    pad_dst = N + (jnp.arange(NW * ppw, dtype=jnp.int32) % NDUM)
    src = jnp.concatenate([edge_index[0].reshape(NW, E // NW),
                           pad_src.reshape(NW, ppw)], axis=1)
    dst = jnp.concatenate([edge_index[1].reshape(NW, E // NW),
                           pad_dst.reshape(NW, ppw)], axis=1)
    zeros = jnp.zeros((NACC, D), jnp.float32)
    batch2 = batch.reshape(N, 1)
    agg = _get_agg()
    h = x
    for i in range(L):
        agg2 = agg(h, src, dst, zeros)
        params = (W1[i], b1[i].reshape(1, D), g1[i].reshape(1, D),
                  beta1[i].reshape(1, D), W2[i], b2[i].reshape(1, D),
                  g2[i].reshape(1, D), beta2[i].reshape(1, D))
        if i < L - 1:
            h = _mlp(h, agg2, *params)
        else:
            out = _mlp_pool(h, agg2, batch2, *params)
    return out


# speedup vs baseline: 3.8800x; 1.0137x over previous
import functools

import jax
import jax.numpy as jnp
from jax import lax
from jax.experimental import pallas as pl
from jax.experimental.pallas import tpu as pltpu
from jax.experimental.pallas import tpu_sc as plsc

N = 10000
E = 320000
D = 128
G = 64
L = 3

NC = 2
NS = 16
NW = NC * NS
CHUNK = 56
NCHUNK = 180
EPW = NCHUNK * CHUNK
NDUM = 16
NACC = N + NDUM
ROWS_PER_SUB = 624
TAIL_OFF = NS * ROWS_PER_SUB
ROWS_TAIL = N - TAIL_OFF
ZTAIL = NACC - TAIL_OFF
EPS = 1e-5

def _agg_impl(h_hbm, src_hbm, dst_hbm, zeros_hbm, out_hbm,
              src_v, dst_v, rows0, rows1, rows2, rows3, acc,
              gsem0, gsem1, gsem2, gsem3, ssem0, ssem1, ssem2, ssem3, zsem):
    c = lax.axis_index("c")
    s = lax.axis_index("s")
    wid = c * NS + s

    zcp = pltpu.async_copy(zeros_hbm.at[pl.ds(s * ROWS_PER_SUB, ROWS_PER_SUB)],
                           acc.at[pl.ds(s * ROWS_PER_SUB, ROWS_PER_SUB)], zsem)
    @pl.when(s == 0)
    def _():
        pltpu.sync_copy(zeros_hbm.at[pl.ds(TAIL_OFF, ZTAIL)],
                        acc.at[pl.ds(TAIL_OFF, ZTAIL)])

    pltpu.sync_copy(src_hbm.at[wid], src_v)
    pltpu.sync_copy(dst_hbm.at[wid], dst_v)

    rows = (rows0, rows1, rows2, rows3)
    gsems = (gsem0, gsem1, gsem2, gsem3)
    ssems = (ssem0, ssem1, ssem2, ssem3)

    def g_desc(j, u):
        idx = src_v.at[pl.ds(j * CHUNK, CHUNK)]
        return pltpu.make_async_copy(h_hbm.at[idx], rows[u], gsems[u])

    def s_desc(j, u):
        idx = dst_v.at[pl.ds(j * CHUNK, CHUNK)]
        return pltpu.make_async_copy(rows[u], acc.at[idx], ssems[u])

    def issue_scatter(j, u):
        idx = dst_v.at[pl.ds(j * CHUNK, CHUNK)]
        pltpu.async_copy(rows[u], acc.at[idx], ssems[u], add=True)

    g_desc(0, 0).start()
    g_desc(1, 1).start()
    zcp.wait()
    plsc.subcore_barrier()

    def body(i, carry):
        for u in range(4):
            j = 4 * i + u
            v = (u + 2) % 4

            @pl.when(j >= 2)
            def _():
                s_desc(j - 2, v).wait()

            @pl.when(j + 2 < NCHUNK)
            def _():
                g_desc(j + 2, v).start()

            g_desc(j, u).wait()
            issue_scatter(j, u)
        return carry

    lax.fori_loop(0, NCHUNK // 4, body, 0)
    s_desc(NCHUNK - 2, 2).wait()
    s_desc(NCHUNK - 1, 3).wait()

    plsc.subcore_barrier()
    pltpu.sync_copy(acc.at[pl.ds(s * ROWS_PER_SUB, ROWS_PER_SUB)],
                    out_hbm.at[pl.ds(c * N + s * ROWS_PER_SUB, ROWS_PER_SUB)])
    @pl.when(s == 0)
    def _():
        pltpu.sync_copy(acc.at[pl.ds(TAIL_OFF, ROWS_TAIL)],
                        out_hbm.at[pl.ds(c * N + TAIL_OFF, ROWS_TAIL)])


@functools.lru_cache(maxsize=None)
def _get_agg():
    mesh = plsc.VectorSubcoreMesh(core_axis_name="c", subcore_axis_name="s")
    return pl.kernel(
        _agg_impl,
        out_type=jax.ShapeDtypeStruct((NC * N, D), jnp.float32),
        mesh=mesh,
        scratch_types=[
            pltpu.VMEM((EPW,), jnp.int32),
            pltpu.VMEM((EPW,), jnp.int32),
            pltpu.VMEM((CHUNK, D), jnp.float32),
            pltpu.VMEM((CHUNK, D), jnp.float32),
            pltpu.VMEM((CHUNK, D), jnp.float32),
            pltpu.VMEM((CHUNK, D), jnp.float32),
            pltpu.VMEM_SHARED((NACC, D), jnp.float32),
        ] + [pltpu.SemaphoreType.DMA] * 9,
    )


def _bn_relu(u, gamma, beta):
    mean = jnp.mean(u, axis=0, keepdims=True)
    var = jnp.mean(jnp.square(u - mean), axis=0, keepdims=True)
    return jnp.maximum((u - mean) * lax.rsqrt(var + EPS) * gamma + beta, 0.0)


def _mlp_common(h_ref, a_ref, w1_ref, b1_ref, g1_ref, be1_ref,
                w2_ref, b2_ref, g2_ref, be2_ref):
    t = h_ref[...] + a_ref[:N] + a_ref[N:]
    u = jnp.dot(t, w1_ref[...], preferred_element_type=jnp.float32) + b1_ref[...]
    u = _bn_relu(u, g1_ref[...], be1_ref[...])
    v = jnp.dot(u, w2_ref[...], preferred_element_type=jnp.float32) + b2_ref[...]
    return _bn_relu(v, g2_ref[...], be2_ref[...])


def _mlp_body(h_ref, a_ref, w1_ref, b1_ref, g1_ref, be1_ref,
              w2_ref, b2_ref, g2_ref, be2_ref, o_ref):
    o_ref[...] = _mlp_common(h_ref, a_ref, w1_ref, b1_ref, g1_ref, be1_ref,
                             w2_ref, b2_ref, g2_ref, be2_ref)


def _mlp_pool_body(h_ref, a_ref, batch_ref, w1_ref, b1_ref, g1_ref, be1_ref,
                   w2_ref, b2_ref, g2_ref, be2_ref, o_ref):
    hout = _mlp_common(h_ref, a_ref, w1_ref, b1_ref, g1_ref, be1_ref,
                       w2_ref, b2_ref, g2_ref, be2_ref)
    gids = lax.broadcasted_iota(jnp.int32, (N, G), 1)
    onehot = (batch_ref[...] == gids).astype(jnp.float32)
    o_ref[...] = lax.dot_general(onehot, hout, (((0,), (0,)), ((), ())),
                                 preferred_element_type=jnp.float32)


_mlp = pl.pallas_call(_mlp_body, out_shape=jax.ShapeDtypeStruct((N, D), jnp.float32))
_mlp_pool = pl.pallas_call(_mlp_pool_body, out_shape=jax.ShapeDtypeStruct((G, D), jnp.float32))


def kernel(x, edge_index, batch, W1, b1, g1, beta1, W2, b2, g2, beta2):
    ppw = EPW - E // NW
    pad_src = (jnp.arange(NW * ppw, dtype=jnp.int32) * 13) % N
    pad_dst = N + (jnp.arange(NW * ppw, dtype=jnp.int32) % NDUM)
    src = jnp.concatenate([edge_index[0].reshape(NW, E // NW),
                           pad_src.reshape(NW, ppw)], axis=1)
    dst = jnp.concatenate([edge_index[1].reshape(NW, E // NW),
                           pad_dst.reshape(NW, ppw)], axis=1)
    zeros = jnp.zeros((NACC, D), jnp.float32)
    batch2 = batch.reshape(N, 1)
    agg = _get_agg()
    h = x
    for i in range(L):
        agg2 = agg(h, src, dst, zeros)
        params = (W1[i], b1[i].reshape(1, D), g1[i].reshape(1, D),
                  beta1[i].reshape(1, D), W2[i], b2[i].reshape(1, D),
                  g2[i].reshape(1, D), beta2[i].reshape(1, D))
        if i < L - 1:
            h = _mlp(h, agg2, *params)
        else:
            out = _mlp_pool(h, agg2, batch2, *params)
    return out


# speedup vs baseline: 4.0993x; 1.0565x over previous
import functools

import jax
import jax.numpy as jnp
from jax import lax
from jax.experimental import pallas as pl
from jax.experimental.pallas import tpu as pltpu
from jax.experimental.pallas import tpu_sc as plsc

N = 10000
E = 320000
D = 128
G = 64
L = 3

NC = 2
NS = 16
NW = NC * NS
CHUNK = 48
NCHUNK = 210
EPW = NCHUNK * CHUNK
NDUM = 16
NACC = N + NDUM
ROWS_PER_SUB = 624
TAIL_OFF = NS * ROWS_PER_SUB
ROWS_TAIL = N - TAIL_OFF
ZTAIL = NACC - TAIL_OFF
EPS = 1e-5

def _agg_impl(h_hbm, src_hbm, dst_hbm, zeros_hbm, out_hbm,
              src_v, dst_v, rows0, rows1, rows2, rows3, rows4, acc,
              gsem0, gsem1, gsem2, gsem3, gsem4,
              ssem0, ssem1, ssem2, ssem3, ssem4, zsem):
    c = lax.axis_index("c")
    s = lax.axis_index("s")
    wid = c * NS + s

    zcp = pltpu.async_copy(zeros_hbm.at[pl.ds(s * ROWS_PER_SUB, ROWS_PER_SUB)],
                           acc.at[pl.ds(s * ROWS_PER_SUB, ROWS_PER_SUB)], zsem)
    @pl.when(s == 0)
    def _():
        pltpu.sync_copy(zeros_hbm.at[pl.ds(TAIL_OFF, ZTAIL)],
                        acc.at[pl.ds(TAIL_OFF, ZTAIL)])

    pltpu.sync_copy(src_hbm.at[wid], src_v)
    pltpu.sync_copy(dst_hbm.at[wid], dst_v)

    rows = (rows0, rows1, rows2, rows3, rows4)
    gsems = (gsem0, gsem1, gsem2, gsem3, gsem4)
    ssems = (ssem0, ssem1, ssem2, ssem3, ssem4)

    def g_desc(j, u):
        idx = src_v.at[pl.ds(j * CHUNK, CHUNK)]
        return pltpu.make_async_copy(h_hbm.at[idx], rows[u], gsems[u])

    def s_desc(j, u):
        idx = dst_v.at[pl.ds(j * CHUNK, CHUNK)]
        return pltpu.make_async_copy(rows[u], acc.at[idx], ssems[u])

    def issue_scatter(j, u):
        idx = dst_v.at[pl.ds(j * CHUNK, CHUNK)]
        pltpu.async_copy(rows[u], acc.at[idx], ssems[u], add=True)

    g_desc(0, 0).start()
    g_desc(1, 1).start()
    g_desc(2, 2).start()
    zcp.wait()
    plsc.subcore_barrier()

    def body(i, carry):
        for u in range(5):
            j = 5 * i + u
            v = (u + 3) % 5

            @pl.when(j >= 2)
            def _():
                s_desc(j - 2, v).wait()

            @pl.when(j + 3 < NCHUNK)
            def _():
                g_desc(j + 3, v).start()

            g_desc(j, u).wait()
            issue_scatter(j, u)
        return carry

    lax.fori_loop(0, NCHUNK // 5, body, 0)
    s_desc(NCHUNK - 2, (NCHUNK - 2) % 5).wait()
    s_desc(NCHUNK - 1, (NCHUNK - 1) % 5).wait()

    plsc.subcore_barrier()
    pltpu.sync_copy(acc.at[pl.ds(s * ROWS_PER_SUB, ROWS_PER_SUB)],
                    out_hbm.at[pl.ds(c * N + s * ROWS_PER_SUB, ROWS_PER_SUB)])
    @pl.when(s == 0)
    def _():
        pltpu.sync_copy(acc.at[pl.ds(TAIL_OFF, ROWS_TAIL)],
                        out_hbm.at[pl.ds(c * N + TAIL_OFF, ROWS_TAIL)])


@functools.lru_cache(maxsize=None)
def _get_agg():
    mesh = plsc.VectorSubcoreMesh(core_axis_name="c", subcore_axis_name="s")
    return pl.kernel(
        _agg_impl,
        out_type=jax.ShapeDtypeStruct((NC * N, D), jnp.float32),
        mesh=mesh,
        scratch_types=[
            pltpu.VMEM((EPW,), jnp.int32),
            pltpu.VMEM((EPW,), jnp.int32),
            pltpu.VMEM((CHUNK, D), jnp.float32),
            pltpu.VMEM((CHUNK, D), jnp.float32),
            pltpu.VMEM((CHUNK, D), jnp.float32),
            pltpu.VMEM((CHUNK, D), jnp.float32),
            pltpu.VMEM((CHUNK, D), jnp.float32),
            pltpu.VMEM_SHARED((NACC, D), jnp.float32),
        ] + [pltpu.SemaphoreType.DMA] * 11,
    )


def _bn_relu(u, gamma, beta):
    mean = jnp.mean(u, axis=0, keepdims=True)
    var = jnp.mean(jnp.square(u - mean), axis=0, keepdims=True)
    return jnp.maximum((u - mean) * lax.rsqrt(var + EPS) * gamma + beta, 0.0)


def _mlp_common(h_ref, a_ref, w1_ref, b1_ref, g1_ref, be1_ref,
                w2_ref, b2_ref, g2_ref, be2_ref):
    t = h_ref[...] + a_ref[:N] + a_ref[N:]
    u = jnp.dot(t, w1_ref[...], preferred_element_type=jnp.float32) + b1_ref[...]
    u = _bn_relu(u, g1_ref[...], be1_ref[...])
    v = jnp.dot(u, w2_ref[...], preferred_element_type=jnp.float32) + b2_ref[...]
    return _bn_relu(v, g2_ref[...], be2_ref[...])


def _mlp_body(h_ref, a_ref, w1_ref, b1_ref, g1_ref, be1_ref,
              w2_ref, b2_ref, g2_ref, be2_ref, o_ref):
    o_ref[...] = _mlp_common(h_ref, a_ref, w1_ref, b1_ref, g1_ref, be1_ref,
                             w2_ref, b2_ref, g2_ref, be2_ref)


def _mlp_pool_body(h_ref, a_ref, batch_ref, w1_ref, b1_ref, g1_ref, be1_ref,
                   w2_ref, b2_ref, g2_ref, be2_ref, o_ref):
    hout = _mlp_common(h_ref, a_ref, w1_ref, b1_ref, g1_ref, be1_ref,
                       w2_ref, b2_ref, g2_ref, be2_ref)
    gids = lax.broadcasted_iota(jnp.int32, (N, G), 1)
    onehot = (batch_ref[...] == gids).astype(jnp.float32)
    o_ref[...] = lax.dot_general(onehot, hout, (((0,), (0,)), ((), ())),
                                 preferred_element_type=jnp.float32)


_mlp = pl.pallas_call(_mlp_body, out_shape=jax.ShapeDtypeStruct((N, D), jnp.float32))
_mlp_pool = pl.pallas_call(_mlp_pool_body, out_shape=jax.ShapeDtypeStruct((G, D), jnp.float32))


def kernel(x, edge_index, batch, W1, b1, g1, beta1, W2, b2, g2, beta2):
    ppw = EPW - E // NW
    pad_src = (jnp.arange(NW * ppw, dtype=jnp.int32) * 13) % N
    pad_dst = N + (jnp.arange(NW * ppw, dtype=jnp.int32) % NDUM)
    src = jnp.concatenate([edge_index[0].reshape(NW, E // NW),
                           pad_src.reshape(NW, ppw)], axis=1)
    dst = jnp.concatenate([edge_index[1].reshape(NW, E // NW),
                           pad_dst.reshape(NW, ppw)], axis=1)
    zeros = jnp.zeros((NACC, D), jnp.float32)
    batch2 = batch.reshape(N, 1)
    agg = _get_agg()
    h = x
    for i in range(L):
        agg2 = agg(h, src, dst, zeros)
        params = (W1[i], b1[i].reshape(1, D), g1[i].reshape(1, D),
                  beta1[i].reshape(1, D), W2[i], b2[i].reshape(1, D),
                  g2[i].reshape(1, D), beta2[i].reshape(1, D))
        if i < L - 1:
            h = _mlp(h, agg2, *params)
        else:
            out = _mlp_pool(h, agg2, batch2, *params)
    return out
